# baseline (speedup 1.0000x reference)
import jax
import jax.numpy as jnp
from jax import lax
from jax.experimental import pallas as pl
from jax.experimental.pallas import tpu as pltpu
from jax.experimental.pallas import tpu_sc as plsc

N = 10000
E = 320000
D = 128
NC = 2
NS = 16
NW = NC * NS
EPW = E // NW
CHUNK = 125
NCHUNK = EPW // CHUNK
RPT = N // NS
L = 16

_mesh = plsc.VectorSubcoreMesh(
    core_axis_name="c", subcore_axis_name="s", num_cores=NC, num_subcores=NS
)


NPAD = 10240
HPT = NPAD // NS


def _deg_body(ei_hbm, out_hbm, src_b, dst_b, ones_v, zbuf, acc_o, acc_i, hsem):
    c = lax.axis_index("c")
    s = lax.axis_index("s")
    wid = s * NC + c

    zeros16 = jnp.zeros((L,), jnp.float32)
    ones16 = jnp.ones((L,), jnp.float32)

    def fill_body(i, carry):
        ones_v[pl.ds(i * L, L)] = ones16
        return carry

    lax.fori_loop(0, 128 // L, fill_body, 0)

    def zero_body(i, carry):
        zbuf[pl.ds(i * L, L)] = zeros16
        return carry

    lax.fori_loop(0, HPT // L, zero_body, 0)
    pltpu.sync_copy(zbuf, acc_o.at[pl.ds(s * HPT, HPT)])
    pltpu.sync_copy(zbuf, acc_i.at[pl.ds(s * HPT, HPT)])
    plsc.subcore_barrier()

    pltpu.sync_copy(ei_hbm.at[0, wid], src_b)
    pltpu.sync_copy(ei_hbm.at[1, wid], dst_b)

    ones = ones_v.at[pl.ds(0, CHUNK)]

    def hist_fire(j, slot):
        pltpu.async_copy(ones, acc_o.at[src_b.at[j]], hsem.at[2 * slot], add=True)
        pltpu.async_copy(ones, acc_i.at[dst_b.at[j]], hsem.at[2 * slot + 1], add=True)

    def hist_wait(j, slot):
        pltpu.make_async_copy(ones, acc_o.at[src_b.at[j]], hsem.at[2 * slot]).wait()
        pltpu.make_async_copy(ones, acc_i.at[dst_b.at[j]], hsem.at[2 * slot + 1]).wait()

    hist_fire(0, 0)
    hist_fire(1, 1)

    def hist_body(jj, carry):
        j0 = jj * 2
        hist_wait(j0, 0)

        @pl.when(j0 + 2 < NCHUNK)
        def _():
            hist_fire(j0 + 2, 0)

        hist_wait(j0 + 1, 1)

        @pl.when(j0 + 3 < NCHUNK)
        def _():
            hist_fire(j0 + 3, 1)

        return carry

    lax.fori_loop(0, NCHUNK // 2, hist_body, 0)
    plsc.subcore_barrier()

    @pl.when(s == 0)
    def _():
        pltpu.sync_copy(acc_o, out_hbm.at[c, 0, 0])
        pltpu.sync_copy(acc_i, out_hbm.at[c, 1, 0])


_deg_call = pl.kernel(
    _deg_body,
    out_type=jax.ShapeDtypeStruct((NC, 2, 1, NPAD), jnp.float32),
    mesh=_mesh,
    scratch_types=[
        pltpu.VMEM((NCHUNK, CHUNK), jnp.int32),
        pltpu.VMEM((NCHUNK, CHUNK), jnp.int32),
        pltpu.VMEM((128,), jnp.float32),
        pltpu.VMEM((HPT,), jnp.float32),
        pltpu.VMEM_SHARED((NPAD,), jnp.float32),
        pltpu.VMEM_SHARED((NPAD,), jnp.float32),
        pltpu.SemaphoreType.DMA((4,)),
    ],
)


DPT = 624
TAIL = N - NS * DPT
ZR = 48
CHK = 80
NCH = EPW // CHK
PH0 = 64
NB = 3


def _edge_body(ei4_hbm, ei_hbm, y_hbm, out_hbm, src_f, dst_b, rows, acc, gsem, ssem):
    c = lax.axis_index("c")
    s = lax.axis_index("s")
    wid = s * NC + c

    zeros16 = jnp.zeros((L,), jnp.float32)

    def zrow(i, carry):
        def zcol(k, carry2):
            rows[0, i, pl.ds(k * L, L)] = zeros16
            return carry2

        return lax.fori_loop(0, D // L, zcol, carry)

    lax.fori_loop(0, ZR, zrow, 0)

    base = s * DPT
    for r in range(DPT // ZR):
        pltpu.sync_copy(rows.at[0].at[pl.ds(0, ZR)], acc.at[pl.ds(base + r * ZR, ZR)])

    @pl.when(s == 0)
    def _():
        pltpu.sync_copy(rows.at[0].at[pl.ds(0, TAIL)], acc.at[pl.ds(NS * DPT, TAIL)])

    plsc.subcore_barrier()

    pltpu.sync_copy(ei4_hbm.at[0, wid, 0], src_f)

    def src_sl(j):
        return src_f.at[pl.ds(pl.multiple_of(j * CHK, CHK), CHK)]

    def g_start(j, bn):
        pltpu.async_copy(y_hbm.at[src_sl(j)], rows.at[bn], gsem.at[bn])

    def g_wait(j, bn):
        pltpu.make_async_copy(y_hbm.at[src_sl(j)], rows.at[bn], gsem.at[bn]).wait()

    def s_start(lj, bn):
        pltpu.async_copy(rows.at[bn], acc.at[dst_b.at[lj]], ssem.at[bn], add=True)

    def s_wait(lj, bn):
        pltpu.make_async_copy(rows.at[bn], acc.at[dst_b.at[lj]], ssem.at[bn]).wait()

    for ph, (B, M) in enumerate([(0, PH0), (PH0, NCH - PH0)]):
        nrows = min(PH0, M)
        pltpu.sync_copy(
            ei_hbm.at[1, wid].at[pl.ds(B, nrows)], dst_b.at[pl.ds(0, nrows)]
        )
        g_start(B + 0, 0)
        g_start(B + 1, 1)

        def lane(l, t):
            g_wait(B + l, t)
            s_start(l, t)

            @pl.when(jnp.logical_and(l + 2 < M, l >= 1))
            def _():
                s_wait(l - 1, (t + 2) % NB)

            @pl.when(l + 2 < M)
            def _():
                g_start(B + l + 2, (t + 2) % NB)

        def group_body(gi, carry):
            for t in range(NB):
                lane(gi * NB + t, t)
            return carry

        ngroups = (M - 1) // NB
        lax.fori_loop(0, ngroups, group_body, 0)
        lp = M - 1
        lane(lp, lp % NB)
        for l in (M - 3, M - 2, M - 1):
            s_wait(l, l % NB)

    plsc.subcore_barrier()

    pltpu.sync_copy(acc.at[pl.ds(base, DPT)], out_hbm.at[c, pl.ds(base, DPT)])

    @pl.when(s == 0)
    def _():
        pltpu.sync_copy(acc.at[pl.ds(NS * DPT, TAIL)], out_hbm.at[c, pl.ds(NS * DPT, TAIL)])


_edge_call = pl.kernel(
    _edge_body,
    out_type=jax.ShapeDtypeStruct((NC, N, D), jnp.float32),
    mesh=_mesh,
    scratch_types=[
        pltpu.VMEM((EPW,), jnp.int32),
        pltpu.VMEM((PH0, CHK), jnp.int32),
        pltpu.VMEM((NB, CHK, D), jnp.float32),
        pltpu.VMEM_SHARED((N, D), jnp.float32),
        pltpu.SemaphoreType.DMA((NB,)),
        pltpu.SemaphoreType.DMA((NB,)),
    ],
)


RB = 1280


def _mm_body(x_ref, w_ref, deg_ref, y_ref):
    deg = deg_ref[0, 0, 0] + deg_ref[1, 0, 0]
    nrm = lax.rsqrt(jnp.maximum(deg, 1.0))
    z = jnp.dot(x_ref[...], w_ref[...], preferred_element_type=jnp.float32)
    y_ref[...] = z * nrm[:, None]


def _ep_body(s_ref, deg_ref, b_ref, o_ref):
    agg = s_ref[0] + s_ref[1]
    deg = deg_ref[0, 1, 0] + deg_ref[1, 1, 0]
    nrm = lax.rsqrt(jnp.maximum(deg, 1.0))
    o_ref[...] = jnp.maximum(agg * nrm[:, None] + b_ref[...], 0.0)


def kernel(x, edge_index, W, b):
    ei_blk = edge_index.reshape(2, NW, NCHUNK, CHUNK)
    ei_blk2 = edge_index.reshape(2, NW, NCH, CHK)
    ei4 = edge_index.reshape(2, NW, 1, EPW)

    degs = _deg_call(ei_blk)
    ngrid = NPAD // RB
    y = pl.pallas_call(
        _mm_body,
        grid=(ngrid,),
        in_specs=[
            pl.BlockSpec((RB, D), lambda i: (i, 0)),
            pl.BlockSpec((D, D), lambda i: (0, 0)),
            pl.BlockSpec((NC, 2, 1, RB), lambda i: (0, 0, 0, i)),
        ],
        out_specs=pl.BlockSpec((RB, D), lambda i: (i, 0)),
        out_shape=jax.ShapeDtypeStruct((N, D), jnp.float32),
    )(x, W, degs)
    parts = _edge_call(ei4, ei_blk2, y)
    out = pl.pallas_call(
        _ep_body,
        grid=(ngrid,),
        in_specs=[
            pl.BlockSpec((NC, RB, D), lambda i: (0, i, 0)),
            pl.BlockSpec((NC, 2, 1, RB), lambda i: (0, 0, 0, i)),
            pl.BlockSpec((1, D), lambda i: (0, 0)),
        ],
        out_specs=pl.BlockSpec((RB, D), lambda i: (i, 0)),
        out_shape=jax.ShapeDtypeStruct((N, D), jnp.float32),
    )(parts, degs, b.reshape(1, D))
    return out


# speedup vs baseline: 1.0286x; 1.0286x over previous
import jax
import jax.numpy as jnp
from jax import lax
from jax.experimental import pallas as pl
from jax.experimental.pallas import tpu as pltpu
from jax.experimental.pallas import tpu_sc as plsc

N = 10000
E = 320000
D = 128
NC = 2
NS = 16
NW = NC * NS
EPW = E // NW
CHUNK = 125
NCHUNK = EPW // CHUNK
RPT = N // NS
L = 16

_mesh = plsc.VectorSubcoreMesh(
    core_axis_name="c", subcore_axis_name="s", num_cores=NC, num_subcores=NS
)


NPAD = 10240
HPT = NPAD // NS


def _deg_body(ei_hbm, out_hbm, src_b, dst_b, ones_v, zbuf, acc_o, acc_i, hsem):
    c = lax.axis_index("c")
    s = lax.axis_index("s")
    wid = s * NC + c

    zeros16 = jnp.zeros((L,), jnp.float32)
    ones16 = jnp.ones((L,), jnp.float32)

    def fill_body(i, carry):
        ones_v[pl.ds(i * L, L)] = ones16
        return carry

    lax.fori_loop(0, 128 // L, fill_body, 0)

    def zero_body(i, carry):
        zbuf[pl.ds(i * L, L)] = zeros16
        return carry

    lax.fori_loop(0, HPT // L, zero_body, 0)
    pltpu.sync_copy(zbuf, acc_o.at[pl.ds(s * HPT, HPT)])
    pltpu.sync_copy(zbuf, acc_i.at[pl.ds(s * HPT, HPT)])
    plsc.subcore_barrier()

    pltpu.sync_copy(ei_hbm.at[0, wid], src_b)
    pltpu.sync_copy(ei_hbm.at[1, wid], dst_b)

    ones = ones_v.at[pl.ds(0, CHUNK)]

    def hist_fire(j, slot):
        pltpu.async_copy(ones, acc_o.at[src_b.at[j]], hsem.at[2 * slot], add=True)
        pltpu.async_copy(ones, acc_i.at[dst_b.at[j]], hsem.at[2 * slot + 1], add=True)

    def hist_wait(j, slot):
        pltpu.make_async_copy(ones, acc_o.at[src_b.at[j]], hsem.at[2 * slot]).wait()
        pltpu.make_async_copy(ones, acc_i.at[dst_b.at[j]], hsem.at[2 * slot + 1]).wait()

    hist_fire(0, 0)
    hist_fire(1, 1)

    def hist_body(jj, carry):
        j0 = jj * 2
        hist_wait(j0, 0)

        @pl.when(j0 + 2 < NCHUNK)
        def _():
            hist_fire(j0 + 2, 0)

        hist_wait(j0 + 1, 1)

        @pl.when(j0 + 3 < NCHUNK)
        def _():
            hist_fire(j0 + 3, 1)

        return carry

    lax.fori_loop(0, NCHUNK // 2, hist_body, 0)
    plsc.subcore_barrier()

    @pl.when(s == 0)
    def _():
        pltpu.sync_copy(acc_o, out_hbm.at[c, 0, 0])
        pltpu.sync_copy(acc_i, out_hbm.at[c, 1, 0])


_deg_call = pl.kernel(
    _deg_body,
    out_type=jax.ShapeDtypeStruct((NC, 2, 1, NPAD), jnp.float32),
    mesh=_mesh,
    scratch_types=[
        pltpu.VMEM((NCHUNK, CHUNK), jnp.int32),
        pltpu.VMEM((NCHUNK, CHUNK), jnp.int32),
        pltpu.VMEM((128,), jnp.float32),
        pltpu.VMEM((HPT,), jnp.float32),
        pltpu.VMEM_SHARED((NPAD,), jnp.float32),
        pltpu.VMEM_SHARED((NPAD,), jnp.float32),
        pltpu.SemaphoreType.DMA((4,)),
    ],
)


DPT = 624
TAIL = N - NS * DPT
ZR = 48
CHK = 80
NCH = EPW // CHK
PH0 = 64
NB = 3


def _edge_body(ei4_hbm, ei_hbm, y_hbm, out_hbm, src_f, dst_b, rows, acc, gsem, ssem):
    c = lax.axis_index("c")
    s = lax.axis_index("s")
    wid = s * NC + c

    zeros16 = jnp.zeros((L,), jnp.float32)

    def zrow(i, carry):
        def zcol(k, carry2):
            rows[0, i, pl.ds(k * L, L)] = zeros16
            return carry2

        return lax.fori_loop(0, D // L, zcol, carry)

    lax.fori_loop(0, ZR, zrow, 0)

    base = s * DPT
    for r in range(DPT // ZR):
        pltpu.sync_copy(rows.at[0].at[pl.ds(0, ZR)], acc.at[pl.ds(base + r * ZR, ZR)])

    @pl.when(s == 0)
    def _():
        pltpu.sync_copy(rows.at[0].at[pl.ds(0, TAIL)], acc.at[pl.ds(NS * DPT, TAIL)])

    plsc.subcore_barrier()

    pltpu.sync_copy(ei4_hbm.at[0, wid, 0], src_f)

    def src_sl(j):
        return src_f.at[pl.ds(pl.multiple_of(j * CHK, CHK), CHK)]

    def g_start(j, bn):
        pltpu.async_copy(y_hbm.at[src_sl(j)], rows.at[bn], gsem.at[bn])

    def g_wait(j, bn):
        pltpu.make_async_copy(y_hbm.at[src_sl(j)], rows.at[bn], gsem.at[bn]).wait()

    def s_start(lj, bn):
        pltpu.async_copy(rows.at[bn], acc.at[dst_b.at[lj]], ssem.at[bn], add=True)

    def s_wait(lj, bn):
        pltpu.make_async_copy(rows.at[bn], acc.at[dst_b.at[lj]], ssem.at[bn]).wait()

    for ph, (B, M) in enumerate([(0, PH0), (PH0, NCH - PH0)]):
        nrows = min(PH0, M)
        pltpu.sync_copy(
            ei_hbm.at[1, wid].at[pl.ds(B, nrows)], dst_b.at[pl.ds(0, nrows)]
        )
        g_start(B + 0, 0)
        g_start(B + 1, 1)

        def lane(l, t):
            g_wait(B + l, t)
            s_start(l, t)

            @pl.when(jnp.logical_and(l + 2 < M, l >= 1))
            def _():
                s_wait(l - 1, (t + 2) % NB)

            @pl.when(l + 2 < M)
            def _():
                g_start(B + l + 2, (t + 2) % NB)

        def group_body(gi, carry):
            for t in range(NB):
                lane(gi * NB + t, t)
            return carry

        ngroups = (M - 1) // NB
        lax.fori_loop(0, ngroups, group_body, 0)
        lp = M - 1
        lane(lp, lp % NB)
        for l in (M - 3, M - 2, M - 1):
            s_wait(l, l % NB)

    plsc.subcore_barrier()

    pltpu.sync_copy(acc.at[pl.ds(base, DPT)], out_hbm.at[c, pl.ds(base, DPT)])

    @pl.when(s == 0)
    def _():
        pltpu.sync_copy(acc.at[pl.ds(NS * DPT, TAIL)], out_hbm.at[c, pl.ds(NS * DPT, TAIL)])


_edge_call = pl.kernel(
    _edge_body,
    out_type=jax.ShapeDtypeStruct((NC, N, D), jnp.float32),
    mesh=_mesh,
    scratch_types=[
        pltpu.VMEM((EPW,), jnp.int32),
        pltpu.VMEM((PH0, CHK), jnp.int32),
        pltpu.VMEM((NB, CHK, D), jnp.float32),
        pltpu.VMEM_SHARED((N, D), jnp.float32),
        pltpu.SemaphoreType.DMA((NB,)),
        pltpu.SemaphoreType.DMA((NB,)),
    ],
)


RB = 1280


def _mm_body(x_ref, w_ref, deg_ref, y_ref):
    deg = (deg_ref[0, 0, 0] + deg_ref[1, 0, 0])[:N]
    nrm = lax.rsqrt(jnp.maximum(deg, 1.0))
    z = jnp.dot(x_ref[...], w_ref[...], preferred_element_type=jnp.float32)
    y_ref[...] = z * nrm[:, None]


def _ep_body(s_ref, deg_ref, b_ref, o_ref):
    agg = s_ref[0] + s_ref[1]
    deg = (deg_ref[0, 1, 0] + deg_ref[1, 1, 0])[:N]
    nrm = lax.rsqrt(jnp.maximum(deg, 1.0))
    o_ref[...] = jnp.maximum(agg * nrm[:, None] + b_ref[...], 0.0)


def kernel(x, edge_index, W, b):
    ei_blk = edge_index.reshape(2, NW, NCHUNK, CHUNK)
    ei_blk2 = edge_index.reshape(2, NW, NCH, CHK)
    ei4 = edge_index.reshape(2, NW, 1, EPW)

    degs = _deg_call(ei_blk)
    y = pl.pallas_call(
        _mm_body, out_shape=jax.ShapeDtypeStruct((N, D), jnp.float32)
    )(x, W, degs)
    parts = _edge_call(ei4, ei_blk2, y)
    out = pl.pallas_call(
        _ep_body, out_shape=jax.ShapeDtypeStruct((N, D), jnp.float32)
    )(parts, degs, b.reshape(1, D))
    return out


# speedup vs baseline: 1.0474x; 1.0183x over previous
import jax
import jax.numpy as jnp
from jax import lax
from jax.experimental import pallas as pl
from jax.experimental.pallas import tpu as pltpu
from jax.experimental.pallas import tpu_sc as plsc

N = 10000
E = 320000
D = 128
NC = 2
NS = 16
NW = NC * NS
EPW = E // NW
CHK = 80
NCH = EPW // CHK
RPT = N // NS
L = 16

_mesh = plsc.VectorSubcoreMesh(
    core_axis_name="c", subcore_axis_name="s", num_cores=NC, num_subcores=NS
)


NPAD = 10240
HPT = NPAD // NS


def _deg_body(ei_hbm, out_hbm, src_b, dst_b, ones_v, zbuf, acc_o, acc_i, hsem):
    c = lax.axis_index("c")
    s = lax.axis_index("s")
    wid = s * NC + c

    zeros16 = jnp.zeros((L,), jnp.float32)
    ones16 = jnp.ones((L,), jnp.float32)

    def fill_body(i, carry):
        ones_v[pl.ds(i * L, L)] = ones16
        return carry

    lax.fori_loop(0, 128 // L, fill_body, 0)

    def zero_body(i, carry):
        zbuf[pl.ds(i * L, L)] = zeros16
        return carry

    lax.fori_loop(0, HPT // L, zero_body, 0)
    pltpu.sync_copy(zbuf, acc_o.at[pl.ds(s * HPT, HPT)])
    pltpu.sync_copy(zbuf, acc_i.at[pl.ds(s * HPT, HPT)])
    plsc.subcore_barrier()

    pltpu.sync_copy(ei_hbm.at[0, wid], src_b)
    pltpu.sync_copy(ei_hbm.at[1, wid], dst_b)

    ones = ones_v.at[pl.ds(0, CHK)]

    def hist_fire(j, slot):
        pltpu.async_copy(ones, acc_o.at[src_b.at[j]], hsem.at[2 * slot], add=True)
        pltpu.async_copy(ones, acc_i.at[dst_b.at[j]], hsem.at[2 * slot + 1], add=True)

    def hist_wait(j, slot):
        pltpu.make_async_copy(ones, acc_o.at[src_b.at[j]], hsem.at[2 * slot]).wait()
        pltpu.make_async_copy(ones, acc_i.at[dst_b.at[j]], hsem.at[2 * slot + 1]).wait()

    hist_fire(0, 0)
    hist_fire(1, 1)

    def hist_body(jj, carry):
        j0 = jj * 2
        hist_wait(j0, 0)

        @pl.when(j0 + 2 < NCH)
        def _():
            hist_fire(j0 + 2, 0)

        hist_wait(j0 + 1, 1)

        @pl.when(j0 + 3 < NCH)
        def _():
            hist_fire(j0 + 3, 1)

        return carry

    lax.fori_loop(0, NCH // 2, hist_body, 0)
    hist_wait(NCH - 1, 0)
    plsc.subcore_barrier()

    @pl.when(s == 0)
    def _():
        pltpu.sync_copy(acc_o, out_hbm.at[c, 0, 0])
        pltpu.sync_copy(acc_i, out_hbm.at[c, 1, 0])


_deg_call = pl.kernel(
    _deg_body,
    out_type=jax.ShapeDtypeStruct((NC, 2, 1, NPAD), jnp.float32),
    mesh=_mesh,
    scratch_types=[
        pltpu.VMEM((NCH, CHK), jnp.int32),
        pltpu.VMEM((NCH, CHK), jnp.int32),
        pltpu.VMEM((128,), jnp.float32),
        pltpu.VMEM((HPT,), jnp.float32),
        pltpu.VMEM_SHARED((NPAD,), jnp.float32),
        pltpu.VMEM_SHARED((NPAD,), jnp.float32),
        pltpu.SemaphoreType.DMA((4,)),
    ],
)


DPT = 624
TAIL = N - NS * DPT
ZR = 48
PH0 = 64
NB = 3


def _edge_body(ei4_hbm, ei_hbm, y_hbm, out_hbm, src_f, dst_b, rows, acc, gsem, ssem):
    c = lax.axis_index("c")
    s = lax.axis_index("s")
    wid = s * NC + c

    zeros16 = jnp.zeros((L,), jnp.float32)

    def zrow(i, carry):
        def zcol(k, carry2):
            rows[0, i, pl.ds(k * L, L)] = zeros16
            return carry2

        return lax.fori_loop(0, D // L, zcol, carry)

    lax.fori_loop(0, ZR, zrow, 0)

    base = s * DPT
    for r in range(DPT // ZR):
        pltpu.sync_copy(rows.at[0].at[pl.ds(0, ZR)], acc.at[pl.ds(base + r * ZR, ZR)])

    @pl.when(s == 0)
    def _():
        pltpu.sync_copy(rows.at[0].at[pl.ds(0, TAIL)], acc.at[pl.ds(NS * DPT, TAIL)])

    plsc.subcore_barrier()

    pltpu.sync_copy(ei4_hbm.at[0, wid, 0], src_f)

    def src_sl(j):
        return src_f.at[pl.ds(pl.multiple_of(j * CHK, CHK), CHK)]

    def g_start(j, bn):
        pltpu.async_copy(y_hbm.at[src_sl(j)], rows.at[bn], gsem.at[bn])

    def g_wait(j, bn):
        pltpu.make_async_copy(y_hbm.at[src_sl(j)], rows.at[bn], gsem.at[bn]).wait()

    def s_start(lj, bn):
        pltpu.async_copy(rows.at[bn], acc.at[dst_b.at[lj]], ssem.at[bn], add=True)

    def s_wait(lj, bn):
        pltpu.make_async_copy(rows.at[bn], acc.at[dst_b.at[lj]], ssem.at[bn]).wait()

    for ph, (B, M) in enumerate([(0, PH0), (PH0, NCH - PH0)]):
        nrows = min(PH0, M)
        pltpu.sync_copy(
            ei_hbm.at[1, wid].at[pl.ds(B, nrows)], dst_b.at[pl.ds(0, nrows)]
        )
        g_start(B + 0, 0)
        g_start(B + 1, 1)

        def lane(l, t):
            g_wait(B + l, t)
            s_start(l, t)

            @pl.when(jnp.logical_and(l + 2 < M, l >= 1))
            def _():
                s_wait(l - 1, (t + 2) % NB)

            @pl.when(l + 2 < M)
            def _():
                g_start(B + l + 2, (t + 2) % NB)

        def group_body(gi, carry):
            for t in range(NB):
                lane(gi * NB + t, t)
            return carry

        ngroups = (M - 1) // NB
        lax.fori_loop(0, ngroups, group_body, 0)
        lp = M - 1
        lane(lp, lp % NB)
        for l in (M - 3, M - 2, M - 1):
            s_wait(l, l % NB)

    plsc.subcore_barrier()

    pltpu.sync_copy(acc.at[pl.ds(base, DPT)], out_hbm.at[c, pl.ds(base, DPT)])

    @pl.when(s == 0)
    def _():
        pltpu.sync_copy(acc.at[pl.ds(NS * DPT, TAIL)], out_hbm.at[c, pl.ds(NS * DPT, TAIL)])


_edge_call = pl.kernel(
    _edge_body,
    out_type=jax.ShapeDtypeStruct((NC, N, D), jnp.float32),
    mesh=_mesh,
    scratch_types=[
        pltpu.VMEM((EPW,), jnp.int32),
        pltpu.VMEM((PH0, CHK), jnp.int32),
        pltpu.VMEM((NB, CHK, D), jnp.float32),
        pltpu.VMEM_SHARED((N, D), jnp.float32),
        pltpu.SemaphoreType.DMA((NB,)),
        pltpu.SemaphoreType.DMA((NB,)),
    ],
)


RB = 1280


def _mm_body(x_ref, w_ref, deg_ref, y_ref):
    deg = (deg_ref[0, 0, 0] + deg_ref[1, 0, 0])[:N]
    nrm = lax.rsqrt(jnp.maximum(deg, 1.0))
    z = jnp.dot(x_ref[...], w_ref[...], preferred_element_type=jnp.float32)
    y_ref[...] = z * nrm[:, None]


def _ep_body(s_ref, deg_ref, b_ref, o_ref):
    agg = s_ref[0] + s_ref[1]
    deg = (deg_ref[0, 1, 0] + deg_ref[1, 1, 0])[:N]
    nrm = lax.rsqrt(jnp.maximum(deg, 1.0))
    o_ref[...] = jnp.maximum(agg * nrm[:, None] + b_ref[...], 0.0)


def kernel(x, edge_index, W, b):
    ei_blk = edge_index.reshape(2, NW, NCH, CHK)
    ei4 = edge_index.reshape(2, NW, 1, EPW)

    degs = _deg_call(ei_blk)
    y = pl.pallas_call(
        _mm_body, out_shape=jax.ShapeDtypeStruct((N, D), jnp.float32)
    )(x, W, degs)
    parts = _edge_call(ei4, ei_blk, y)
    out = pl.pallas_call(
        _ep_body, out_shape=jax.ShapeDtypeStruct((N, D), jnp.float32)
    )(parts, degs, b.reshape(1, D))
    return out


# speedup vs baseline: 1.0481x; 1.0007x over previous
import jax
import jax.numpy as jnp
from jax import lax
from jax.experimental import pallas as pl
from jax.experimental.pallas import tpu as pltpu
from jax.experimental.pallas import tpu_sc as plsc

N = 10000
E = 320000
D = 128
NC = 2
NS = 16
NW = NC * NS
EPW = E // NW
CHK = 80
NCH = EPW // CHK
RPT = N // NS
L = 16

_mesh = plsc.VectorSubcoreMesh(
    core_axis_name="c", subcore_axis_name="s", num_cores=NC, num_subcores=NS
)


NPAD = 10240
HPT = NPAD // NS


def _deg_body(ei_hbm, out_hbm, src_b, dst_b, ones_v, zbuf, acc_o, acc_i, hsem):
    c = lax.axis_index("c")
    s = lax.axis_index("s")
    wid = s * NC + c

    zeros16 = jnp.zeros((L,), jnp.float32)
    ones16 = jnp.ones((L,), jnp.float32)

    def fill_body(i, carry):
        ones_v[pl.ds(i * L, L)] = ones16
        return carry

    lax.fori_loop(0, 128 // L, fill_body, 0)

    def zero_body(i, carry):
        zbuf[pl.ds(i * L, L)] = zeros16
        return carry

    lax.fori_loop(0, HPT // L, zero_body, 0)
    pltpu.sync_copy(zbuf, acc_o.at[pl.ds(s * HPT, HPT)])
    pltpu.sync_copy(zbuf, acc_i.at[pl.ds(s * HPT, HPT)])
    plsc.subcore_barrier()

    pltpu.sync_copy(ei_hbm.at[0, wid], src_b)
    pltpu.sync_copy(ei_hbm.at[1, wid], dst_b)

    ones = ones_v.at[pl.ds(0, CHK)]

    def hist_fire(j, slot):
        pltpu.async_copy(ones, acc_o.at[src_b.at[j]], hsem.at[2 * slot], add=True)
        pltpu.async_copy(ones, acc_i.at[dst_b.at[j]], hsem.at[2 * slot + 1], add=True)

    def hist_wait(j, slot):
        pltpu.make_async_copy(ones, acc_o.at[src_b.at[j]], hsem.at[2 * slot]).wait()
        pltpu.make_async_copy(ones, acc_i.at[dst_b.at[j]], hsem.at[2 * slot + 1]).wait()

    hist_fire(0, 0)
    hist_fire(1, 1)

    def hist_body(jj, carry):
        j0 = jj * 2
        hist_wait(j0, 0)

        @pl.when(j0 + 2 < NCH)
        def _():
            hist_fire(j0 + 2, 0)

        hist_wait(j0 + 1, 1)

        @pl.when(j0 + 3 < NCH)
        def _():
            hist_fire(j0 + 3, 1)

        return carry

    lax.fori_loop(0, NCH // 2, hist_body, 0)
    hist_wait(NCH - 1, 0)
    plsc.subcore_barrier()

    @pl.when(s == 0)
    def _():
        pltpu.sync_copy(acc_o, out_hbm.at[c, 0, 0])
        pltpu.sync_copy(acc_i, out_hbm.at[c, 1, 0])


_deg_call = pl.kernel(
    _deg_body,
    out_type=jax.ShapeDtypeStruct((NC, 2, 1, NPAD), jnp.float32),
    mesh=_mesh,
    scratch_types=[
        pltpu.VMEM((NCH, CHK), jnp.int32),
        pltpu.VMEM((NCH, CHK), jnp.int32),
        pltpu.VMEM((128,), jnp.float32),
        pltpu.VMEM((HPT,), jnp.float32),
        pltpu.VMEM_SHARED((NPAD,), jnp.float32),
        pltpu.VMEM_SHARED((NPAD,), jnp.float32),
        pltpu.SemaphoreType.DMA((4,)),
    ],
)


DPT = 624
TAIL = N - NS * DPT
ZR = 48
PH0 = 64
NB = 3


def _edge_body(ei4_hbm, ei_hbm, y_hbm, out_hbm, src_f, dst_b, rows, acc, gsem, ssem):
    c = lax.axis_index("c")
    s = lax.axis_index("s")
    wid = s * NC + c

    zeros16 = jnp.zeros((L,), jnp.float32)

    def zrow(i, carry):
        def zcol(k, carry2):
            rows[0, i, pl.ds(k * L, L)] = zeros16
            return carry2

        return lax.fori_loop(0, D // L, zcol, carry)

    lax.fori_loop(0, ZR, zrow, 0)

    base = s * DPT
    for r in range(DPT // ZR):
        pltpu.sync_copy(rows.at[0].at[pl.ds(0, ZR)], acc.at[pl.ds(base + r * ZR, ZR)])

    @pl.when(s == 0)
    def _():
        pltpu.sync_copy(rows.at[0].at[pl.ds(0, TAIL)], acc.at[pl.ds(NS * DPT, TAIL)])

    plsc.subcore_barrier()

    pltpu.sync_copy(ei4_hbm.at[0, wid, 0], src_f)

    def src_sl(j):
        return src_f.at[pl.ds(pl.multiple_of(j * CHK, CHK), CHK)]

    def g_start(j, bn):
        pltpu.async_copy(y_hbm.at[src_sl(j)], rows.at[bn], gsem.at[bn])

    def g_wait(j, bn):
        pltpu.make_async_copy(y_hbm.at[src_sl(j)], rows.at[bn], gsem.at[bn]).wait()

    def s_start(lj, bn):
        pltpu.async_copy(rows.at[bn], acc.at[dst_b.at[lj]], ssem.at[bn], add=True)

    def s_wait(lj, bn):
        pltpu.make_async_copy(rows.at[bn], acc.at[dst_b.at[lj]], ssem.at[bn]).wait()

    for ph, (B, M) in enumerate([(0, PH0), (PH0, NCH - PH0)]):
        nrows = min(PH0, M)
        pltpu.sync_copy(
            ei_hbm.at[1, wid].at[pl.ds(B, nrows)], dst_b.at[pl.ds(0, nrows)]
        )
        g_start(B + 0, 0)
        g_start(B + 1, 1)

        def lane(l, t):
            g_wait(B + l, t)
            s_start(l, t)

            @pl.when(jnp.logical_and(l + 2 < M, l >= 1))
            def _():
                s_wait(l - 1, (t + 2) % NB)

            @pl.when(l + 2 < M)
            def _():
                g_start(B + l + 2, (t + 2) % NB)

        def group_body(gi, carry):
            for t in range(NB):
                lane(gi * NB + t, t)
            return carry

        ngroups = (M - 1) // NB
        lax.fori_loop(0, ngroups, group_body, 0)
        lp = M - 1
        lane(lp, lp % NB)
        for l in (M - 3, M - 2, M - 1):
            s_wait(l, l % NB)

    plsc.subcore_barrier()

    pltpu.sync_copy(acc.at[pl.ds(base, DPT)], out_hbm.at[c, pl.ds(base, DPT)])

    @pl.when(s == 0)
    def _():
        pltpu.sync_copy(acc.at[pl.ds(NS * DPT, TAIL)], out_hbm.at[c, pl.ds(NS * DPT, TAIL)])


_edge_call = pl.kernel(
    _edge_body,
    out_type=jax.ShapeDtypeStruct((NC, N, D), jnp.float32),
    mesh=_mesh,
    scratch_types=[
        pltpu.VMEM((EPW,), jnp.int32),
        pltpu.VMEM((PH0, CHK), jnp.int32),
        pltpu.VMEM((NB, CHK, D), jnp.float32),
        pltpu.VMEM_SHARED((N, D), jnp.float32),
        pltpu.SemaphoreType.DMA((NB,)),
        pltpu.SemaphoreType.DMA((NB,)),
    ],
)


def _mm_body(x_ref, w_ref, deg_ref, y_ref):
    deg = (deg_ref[0, 0, 0] + deg_ref[1, 0, 0])[:N]
    nrm = lax.rsqrt(jnp.maximum(deg, 1.0))
    z = jnp.dot(x_ref[...], w_ref[...], preferred_element_type=jnp.float32)
    y_ref[...] = z * nrm[:, None]


def _ep_body(s_ref, deg_ref, b_ref, o_ref):
    agg = s_ref[0] + s_ref[1]
    deg = (deg_ref[0, 1, 0] + deg_ref[1, 1, 0])[:N]
    nrm = lax.rsqrt(jnp.maximum(deg, 1.0))
    o_ref[...] = jnp.maximum(agg * nrm[:, None] + b_ref[...], 0.0)


def kernel(x, edge_index, W, b):
    ei_blk = edge_index.reshape(2, NW, NCH, CHK)
    ei4 = edge_index.reshape(2, NW, 1, EPW)

    degs = _deg_call(ei_blk)
    y = pl.pallas_call(
        _mm_body, out_shape=jax.ShapeDtypeStruct((N, D), jnp.float32)
    )(x, W, degs)
    parts = _edge_call(ei4, ei_blk, y)
    out = pl.pallas_call(
        _ep_body, out_shape=jax.ShapeDtypeStruct((N, D), jnp.float32)
    )(parts, degs, b.reshape(1, D))
    return out
